# baseline (speedup 1.0000x reference)
import jax
import jax.numpy as jnp
from jax.experimental import pallas as pl


def _vq_block(z_ref, w_ref, sumsq_ref, outq_ref, enc_ref, inds_ref):
    b = pl.program_id(0)
    K = w_ref.shape[0]
    x_ct = z_ref[0]
    x_tk = x_ct.T
    w = w_ref[...]
    T = x_tk.shape[0]

    mm = jnp.dot(x_tk, w.T, preferred_element_type=jnp.float32)
    x2 = jnp.sum(x_tk * x_tk, axis=1, keepdims=True)
    w2 = jnp.sum(w * w, axis=1)
    dist = (x2 + w2[None, :]) - 2.0 * mm

    minv = jnp.min(dist, axis=1, keepdims=True)
    iota_k = jax.lax.broadcasted_iota(jnp.int32, (T, K), 1)
    idx = jnp.min(jnp.where(dist == minv, iota_k, K), axis=1)

    onehot = (iota_k == idx[:, None]).astype(jnp.float32)
    q = jnp.dot(onehot, w, preferred_element_type=jnp.float32)

    st = x_tk + (q - x_tk)
    outq_ref[0] = st.T
    enc_ref[...] = onehot
    inds_ref[0, 0, :] = idx

    diff = q - x_tk
    part = jnp.sum(diff * diff).reshape(1, 1)

    @pl.when(b == 0)
    def _():
        sumsq_ref[...] = part

    @pl.when(b != 0)
    def _():
        sumsq_ref[...] += part


def kernel(z_e_x, weight):
    B, C, H, W = z_e_x.shape
    K, D = weight.shape
    HW = H * W
    zr = z_e_x.reshape(B, C, HW)

    sumsq, outq, enc, inds = pl.pallas_call(
        _vq_block,
        grid=(B,),
        in_specs=[
            pl.BlockSpec((1, C, HW), lambda b: (b, 0, 0)),
            pl.BlockSpec((K, D), lambda b: (0, 0)),
        ],
        out_specs=[
            pl.BlockSpec((1, 1), lambda b: (0, 0)),
            pl.BlockSpec((1, C, HW), lambda b: (b, 0, 0)),
            pl.BlockSpec((HW, K), lambda b: (b, 0)),
            pl.BlockSpec((1, 1, HW), lambda b: (b, 0, 0)),
        ],
        out_shape=[
            jax.ShapeDtypeStruct((1, 1), jnp.float32),
            jax.ShapeDtypeStruct((B, C, HW), jnp.float32),
            jax.ShapeDtypeStruct((B * HW, K), jnp.float32),
            jax.ShapeDtypeStruct((B, 1, HW), jnp.int32),
        ],
    )(zr, weight)

    loss = sumsq[0, 0] * (2.0 / (B * HW * C))
    return (loss, outq.reshape(B, C, H, W), enc, inds.reshape(B * HW))
